# baseline (speedup 1.0000x reference)
import functools

import jax
import jax.numpy as jnp
from jax import lax
from jax.experimental import pallas as pl
from jax.experimental.pallas import tpu as pltpu
from jax.experimental.pallas import tpu_sc as plsc

B = 16384
D = 64
NW = 32
BPW = B // NW
CH = 128
NCHUNK = BPW // CH
L = 16


def _transe_body(heads_hbm, rels_hbm, tails_hbm, ent_hbm, rel_hbm, out_hbm,
                 hidx, ridx, tidx, hrows, rrows, trows, outv, sem):
    wid = lax.axis_index("s") * 2 + lax.axis_index("c")

    pltpu.sync_copy(heads_hbm.at[pl.ds(wid * NCHUNK, NCHUNK)], hidx)
    pltpu.sync_copy(rels_hbm.at[pl.ds(wid * NCHUNK, NCHUNK)], ridx)
    pltpu.sync_copy(tails_hbm.at[pl.ds(wid * NCHUNK, NCHUNK)], tidx)

    copies = []
    for c in range(NCHUNK):
        dst = pl.ds(c * CH, CH)
        copies.append(pltpu.async_copy(ent_hbm.at[hidx.at[c]], hrows.at[dst], sem))
        copies.append(pltpu.async_copy(rel_hbm.at[ridx.at[c]], rrows.at[dst], sem))
        copies.append(pltpu.async_copy(ent_hbm.at[tidx.at[c]], trows.at[dst], sem))
    for cp in copies:
        cp.wait()

    lane = lax.iota(jnp.int32, L)

    def group_body(g, carry):
        base = g * L
        svec = jnp.zeros((L,), jnp.float32)
        for k in range(L):
            i = base + k
            acc = jnp.zeros((L,), jnp.float32)
            for j in range(D // L):
                sl = pl.ds(j * L, L)
                h = hrows[i, sl]
                r = rrows[i, sl]
                t = trows[i, sl]
                d = (h + r) - t
                acc = acc + d * d
            svec = jnp.where(lane == k, jnp.sum(acc), svec)
        x = svec + 1e-12
        xi = plsc.bitcast(x, jnp.int32)
        yi = jnp.full((L,), 0x5F3759DF, jnp.int32) - (xi >> 1)
        y = plsc.bitcast(yi, jnp.float32)
        for _ in range(3):
            y = y * (1.5 - 0.5 * x * y * y)
        outv[pl.ds(base, L)] = -(x * y)
        return carry

    lax.fori_loop(0, BPW // L, group_body, 0)

    pltpu.sync_copy(outv, out_hbm.at[pl.ds(wid * BPW, BPW)])


@jax.jit
def _transe_sc(heads2, rels2, tails2, entity_embed, relation_embed):
    mesh = plsc.VectorSubcoreMesh(core_axis_name="c", subcore_axis_name="s")
    return pl.kernel(
        _transe_body,
        mesh=mesh,
        compiler_params=pltpu.CompilerParams(
            needs_layout_passes=False, use_tc_tiling_on_sc=False),
        out_type=jax.ShapeDtypeStruct((B,), jnp.float32),
        scratch_types=[
            pltpu.VMEM((NCHUNK, CH), jnp.int32),
            pltpu.VMEM((NCHUNK, CH), jnp.int32),
            pltpu.VMEM((NCHUNK, CH), jnp.int32),
            pltpu.VMEM((BPW, D), jnp.float32),
            pltpu.VMEM((BPW, D), jnp.float32),
            pltpu.VMEM((BPW, D), jnp.float32),
            pltpu.VMEM((BPW,), jnp.float32),
            pltpu.SemaphoreType.DMA,
        ],
    )(heads2, rels2, tails2, entity_embed, relation_embed)


def kernel(heads, relations, tails, entity_embed, relation_embed):
    heads2 = heads.astype(jnp.int32).reshape(B // CH, CH)
    rels2 = relations.astype(jnp.int32).reshape(B // CH, CH)
    tails2 = tails.astype(jnp.int32).reshape(B // CH, CH)
    return _transe_sc(heads2, rels2, tails2, entity_embed, relation_embed)
